# baseline (speedup 1.0000x reference)
import functools

import jax
import jax.numpy as jnp
from jax import lax
from jax.experimental import pallas as pl
from jax.experimental.pallas import tpu as pltpu
from jax.experimental.pallas import tpu_sc as plsc

_N, _E, _D = 10000, 320000, 128
_NC, _NS = 2, 16
_NW = _NC * _NS
_EPW = _E // _NW
_CH = 80
_NCHUNK = _EPW // _CH
_NP = 10240
_RPT = _NP // _NS
_CHS = 40
_NCHS = _EPW // _CHS
_BE = 4000
_GE = _E // _BE
_BN = 2000

@functools.lru_cache(maxsize=1)
def _sc_mesh():
    return plsc.VectorSubcoreMesh(
        core_axis_name="c", subcore_axis_name="s",
        num_cores=_NC, num_subcores=_NS)


def _node_mm_body(h_ref, w_ref, qh_ref, rh_ref, vh_ref, uh_ref):
    p = jnp.dot(h_ref[...], w_ref[...], preferred_element_type=jnp.float32)
    qh_ref[...] = p[:, 0 * _D:1 * _D]
    rh_ref[...] = p[:, 1 * _D:2 * _D]
    vh_ref[...] = p[:, 2 * _D:3 * _D]
    uh_ref[...] = p[:, 3 * _D:4 * _D]


_node_mm = pl.pallas_call(
    _node_mm_body,
    grid=(_N // _BN,),
    in_specs=[
        pl.BlockSpec((_BN, _D), lambda i: (i, 0)),
        pl.BlockSpec((_D, 4 * _D), lambda i: (0, 0)),
    ],
    out_specs=[pl.BlockSpec((_BN, _D), lambda i: (i, 0))] * 4,
    out_shape=[jax.ShapeDtypeStruct((_N, _D), jnp.float32)] * 4,
)


def _edge1_body(pT_ref, e_ref, g1_ref, ehat_ref, stats_ref):
    e_hat = (jnp.dot(e_ref[...], pT_ref[...], preferred_element_type=jnp.float32)
             + g1_ref[...])
    ehat_ref[...] = e_hat
    s1 = jnp.sum(e_hat, axis=0, keepdims=True)
    s2 = jnp.sum(e_hat * e_hat, axis=0, keepdims=True)
    blk = jnp.concatenate([s1, s2, jnp.zeros((6, _D), jnp.float32)], axis=0)

    @pl.when(pl.program_id(0) == 0)
    def _():
        stats_ref[...] = blk

    @pl.when(pl.program_id(0) != 0)
    def _():
        stats_ref[...] += blk


_edge_pass1 = pl.pallas_call(
    _edge1_body,
    grid=(_GE,),
    in_specs=[
        pl.BlockSpec((_D, _D), lambda i: (0, 0)),
        pl.BlockSpec((_BE, _D), lambda i: (i, 0)),
        pl.BlockSpec((_BE, _D), lambda i: (i, 0)),
    ],
    out_specs=[
        pl.BlockSpec((_BE, _D), lambda i: (i, 0)),
        pl.BlockSpec((8, _D), lambda i: (0, 0)),
    ],
    out_shape=[
        jax.ShapeDtypeStruct((_E, _D), jnp.float32),
        jax.ShapeDtypeStruct((8, _D), jnp.float32),
    ],
)


def _edge2_body(stats_ref, m1T_ref, m2T_ref, b1_ref, b2_ref, g_ref, bta_ref,
                e_ref, ehat_ref, enew_ref):
    mean = stats_ref[0:1, :] * (1.0 / _E)
    var = stats_ref[1:2, :] * (1.0 / _E) - mean * mean
    rstd = lax.rsqrt(var + 1e-5)
    x = (ehat_ref[...] - mean) * rstd * g_ref[...] + bta_ref[...]
    t = jnp.maximum(
        jnp.dot(x, m1T_ref[...], preferred_element_type=jnp.float32)
        + b1_ref[...], 0.0)
    y = jnp.dot(t, m2T_ref[...], preferred_element_type=jnp.float32) + b2_ref[...]
    enew_ref[...] = e_ref[...] + y


_edge_pass2 = pl.pallas_call(
    _edge2_body,
    grid=(_GE,),
    in_specs=[
        pl.BlockSpec((8, _D), lambda i: (0, 0)),
        pl.BlockSpec((_D, _D), lambda i: (0, 0)),
        pl.BlockSpec((_D, _D), lambda i: (0, 0)),
        pl.BlockSpec((1, _D), lambda i: (0, 0)),
        pl.BlockSpec((1, _D), lambda i: (0, 0)),
        pl.BlockSpec((1, _D), lambda i: (0, 0)),
        pl.BlockSpec((1, _D), lambda i: (0, 0)),
        pl.BlockSpec((_BE, _D), lambda i: (i, 0)),
        pl.BlockSpec((_BE, _D), lambda i: (i, 0)),
    ],
    out_specs=pl.BlockSpec((_BE, _D), lambda i: (i, 0)),
    out_shape=jax.ShapeDtypeStruct((_E, _D), jnp.float32),
)


def _node_bn_body(h_ref, uh_ref, a0_ref, a1_ref, g_ref, bta_ref, hnew_ref):
    z = uh_ref[...] + a0_ref[...] + a1_ref[...]
    mu = jnp.mean(z, axis=0, keepdims=True)
    var = jnp.mean((z - mu) * (z - mu), axis=0, keepdims=True)
    zn = (z - mu) * lax.rsqrt(var + 1e-5) * g_ref[...] + bta_ref[...]
    hnew_ref[...] = h_ref[...] + zn


_node_pass = pl.pallas_call(
    _node_bn_body,
    out_shape=jax.ShapeDtypeStruct((_N, _D), jnp.float32),
)


@functools.lru_cache(maxsize=1)
def _make_gather():
    @functools.partial(
        pl.kernel,
        mesh=_sc_mesh(),
        out_type=jax.ShapeDtypeStruct((_E, _D), jnp.float32),
        scratch_types=[
            pltpu.VMEM((_NCHUNK, _CH), jnp.int32),
            pltpu.VMEM((_NCHUNK, _CH), jnp.int32),
            pltpu.VMEM((_CH, _D), jnp.float32),
            pltpu.VMEM((_CH, _D), jnp.float32),
            pltpu.VMEM((_CH, _D), jnp.float32),
            pltpu.VMEM((_CH, _D), jnp.float32),
            pltpu.SemaphoreType.DMA,
            pltpu.SemaphoreType.DMA,
            pltpu.SemaphoreType.DMA,
            pltpu.SemaphoreType.DMA,
            pltpu.SemaphoreType.DMA,
            pltpu.SemaphoreType.DMA,
        ],
    )
    def _gather(qh, rh, src3, dst3, g1,
                src_i, dst_i, qb0, qb1, rb0, rb1,
                gq0, gq1, gr0, gr1, os0, os1):
        wid = lax.axis_index("s") * _NC + lax.axis_index("c")
        base0 = wid * _EPW
        qbs, rbs = (qb0, qb1), (rb0, rb1)
        gqs, grs, oss = (gq0, gq1), (gr0, gr1), (os0, os1)
        pltpu.sync_copy(src3.at[wid], src_i)
        pltpu.sync_copy(dst3.at[wid], dst_i)
        pltpu.async_copy(qh.at[src_i.at[0]], qbs[0], gqs[0])
        pltpu.async_copy(rh.at[dst_i.at[0]], rbs[0], grs[0])

        @pl.loop(0, _NCHUNK + 1, step=2)
        def pair(ii):
            for b in range(2):
                i = ii + b

                @pl.when(i < _NCHUNK)
                def _():
                    nb = 1 - b
                    @pl.when(i > 0)
                    def _():
                        pltpu.make_async_copy(
                            qbs[nb], g1.at[pl.ds(base0, _CH)], oss[nb]).wait()

                    @pl.when(i + 1 < _NCHUNK)
                    def _():
                        pltpu.async_copy(qh.at[src_i.at[i + 1]], qbs[nb],
                                         gqs[nb])
                        pltpu.async_copy(rh.at[dst_i.at[i + 1]], rbs[nb],
                                         grs[nb])

                    pltpu.make_async_copy(qh.at[src_i.at[i]], qbs[b],
                                          gqs[b]).wait()
                    pltpu.make_async_copy(rh.at[dst_i.at[i]], rbs[b],
                                          grs[b]).wait()

                    @pl.loop(0, _CH, unroll=4)
                    def row(r):
                        for g in range(_D // 16):
                            sl = pl.ds(g * 16, 16)
                            qbs[b][r, sl] = qbs[b][r, sl] + rbs[b][r, sl]

                    pltpu.async_copy(
                        qbs[b], g1.at[pl.ds(base0 + i * _CH, _CH)], oss[b])

        pltpu.make_async_copy(qbs[0], g1.at[pl.ds(base0, _CH)], oss[0]).wait()

    return _gather


@functools.lru_cache(maxsize=1)
def _make_scatter():
    @functools.partial(
        pl.kernel,
        mesh=_sc_mesh(),
        out_type=jax.ShapeDtypeStruct((_NC, _NP, _D), jnp.float32),
        scratch_types=[
            pltpu.VMEM((_CHS,), jnp.int32),
            pltpu.VMEM((_CHS,), jnp.int32),
            pltpu.VMEM((_CHS,), jnp.int32),
            pltpu.VMEM((_CHS,), jnp.int32),
            pltpu.VMEM((_CHS, _D), jnp.float32),
            pltpu.VMEM((_CHS, _D), jnp.float32),
            pltpu.VMEM((_CHS, _D), jnp.float32),
            pltpu.VMEM((_CHS, _D), jnp.float32),
            pltpu.VMEM_SHARED((_NP, _D), jnp.float32),
            pltpu.SemaphoreType.DMA,
            pltpu.SemaphoreType.DMA,
            pltpu.SemaphoreType.DMA,
            pltpu.SemaphoreType.DMA,
            pltpu.SemaphoreType.DMA,
            pltpu.SemaphoreType.DMA,
            pltpu.SemaphoreType.DMA,
            pltpu.SemaphoreType.DMA,
        ],
    )
    def _scatter(ehat, vh, srcf, dstf, zeros_hbm, out,
                 sv0, sv1, dv0, dv1, eb0, eb1, vb0, vb1, agg_sh,
                 is0, is1, gv0, gv1, ge0, ge1, ss0, ss1):
        c = lax.axis_index("c")
        s = lax.axis_index("s")
        wid = s * _NC + c
        svs, dvs = (sv0, sv1), (dv0, dv1)
        ebs, vbs = (eb0, eb1), (vb0, vb1)
        iss, gvs, ges = (is0, is1), (gv0, gv1), (ge0, ge1)
        sss = (ss0, ss1)
        pltpu.sync_copy(zeros_hbm.at[pl.ds(s * _RPT, _RPT)],
                        agg_sh.at[pl.ds(s * _RPT, _RPT)])
        base0 = wid * _EPW
        pltpu.sync_copy(srcf.at[pl.ds(base0, _CHS)], svs[0])
        pltpu.sync_copy(dstf.at[pl.ds(base0, _CHS)], dvs[0])
        pltpu.async_copy(vh.at[dvs[0]], vbs[0], gvs[0])
        pltpu.async_copy(ehat.at[pl.ds(base0, _CHS)], ebs[0], ges[0])
        plsc.subcore_barrier()

        @pl.loop(0, _NCHS + 1, step=2)
        def pair(ii):
            for b in range(2):
                i = ii + b

                @pl.when(i < _NCHS)
                def _():
                    nb = 1 - b
                    @pl.when(i > 0)
                    def _():
                        pltpu.make_async_copy(
                            vbs[nb], agg_sh.at[svs[nb]], sss[nb]).wait()

                    @pl.when(i + 1 < _NCHS)
                    def _():
                        pltpu.async_copy(
                            srcf.at[pl.ds(base0 + (i + 1) * _CHS, _CHS)],
                            svs[nb], iss[nb])
                        pltpu.async_copy(
                            dstf.at[pl.ds(base0 + (i + 1) * _CHS, _CHS)],
                            dvs[nb], iss[nb])

                    pltpu.make_async_copy(vh.at[dvs[b]], vbs[b],
                                          gvs[b]).wait()
                    pltpu.make_async_copy(ehat.at[pl.ds(base0, _CHS)], ebs[b],
                                          ges[b]).wait()

                    @pl.when(i + 1 < _NCHS)
                    def _():
                        pltpu.make_async_copy(
                            srcf.at[pl.ds(base0, _CHS)], svs[nb],
                            iss[nb]).wait()
                        pltpu.make_async_copy(
                            dstf.at[pl.ds(base0, _CHS)], dvs[nb],
                            iss[nb]).wait()
                        pltpu.async_copy(vh.at[dvs[nb]], vbs[nb], gvs[nb])
                        pltpu.async_copy(
                            ehat.at[pl.ds(base0 + (i + 1) * _CHS, _CHS)],
                            ebs[nb], ges[nb])

                    @pl.loop(0, _CHS, unroll=4)
                    def row(r):
                        for g in range(_D // 16):
                            sl = pl.ds(g * 16, 16)
                            x = ebs[b][r, sl]
                            gate = 1.0 / (1.0 + jnp.exp(-x))
                            vbs[b][r, sl] = gate * vbs[b][r, sl]

                    pltpu.async_copy(vbs[b], agg_sh.at[svs[b]], sss[b],
                                     add=True)

        pltpu.make_async_copy(vbs[1], agg_sh.at[svs[1]], sss[1]).wait()
        plsc.subcore_barrier()
        pltpu.sync_copy(agg_sh.at[pl.ds(s * _RPT, _RPT)],
                        out.at[c, pl.ds(s * _RPT, _RPT)])

    return _scatter


def kernel(h, e, edge_index, P_w, Q_w, R_w, U_w, V_w,
           m1_w, m1_b, m2_w, m2_b, e_gamma, e_beta, n_gamma, n_beta):
    src3 = edge_index[0].reshape(_NW, _NCHUNK, _CH)
    dst3 = edge_index[1].reshape(_NW, _NCHUNK, _CH)
    w4 = jnp.concatenate([Q_w.T, R_w.T, V_w.T, U_w.T], axis=1)
    qh, rh, vh, uh = _node_mm(h, w4)
    g1 = _make_gather()(qh, rh, src3, dst3)
    ehat, stats = _edge_pass1(P_w.T, e, g1)
    aggc = _make_scatter()(ehat, vh, edge_index[0], edge_index[1],
                           jnp.zeros((_NP, _D), jnp.float32))
    e_new = _edge_pass2(stats, m1_w.T, m2_w.T,
                        m1_b.reshape(1, _D), m2_b.reshape(1, _D),
                        e_gamma.reshape(1, _D), e_beta.reshape(1, _D),
                        e, ehat)
    h_new = _node_pass(h, uh, aggc[0, :_N], aggc[1, :_N],
                       n_gamma.reshape(1, _D), n_beta.reshape(1, _D))
    return (h_new, e_new)


# speedup vs baseline: 2.0306x; 2.0306x over previous
import functools

import jax
import jax.numpy as jnp
from jax import lax
from jax.experimental import pallas as pl
from jax.experimental.pallas import tpu as pltpu
from jax.experimental.pallas import tpu_sc as plsc

_N, _E, _D = 10000, 320000, 128
_NC, _NS = 2, 16
_NW = _NC * _NS
_EPW = _E // _NW
_CH = 80
_NCHUNK = _EPW // _CH
_NP = 10240
_RPT = _NP // _NS
_CHS = 40
_NCHS = _EPW // _CHS
_BE = 4000
_GE = _E // _BE
_BN = 2000

@functools.lru_cache(maxsize=1)
def _sc_mesh():
    return plsc.VectorSubcoreMesh(
        core_axis_name="c", subcore_axis_name="s",
        num_cores=_NC, num_subcores=_NS)


def _node_mm_body(h_ref, w_ref, qh_ref, rh_ref, vh_ref, uh_ref):
    p = jnp.dot(h_ref[...], w_ref[...], preferred_element_type=jnp.float32)
    qh_ref[...] = p[:, 0 * _D:1 * _D]
    rh_ref[...] = p[:, 1 * _D:2 * _D]
    vh_ref[...] = p[:, 2 * _D:3 * _D]
    uh_ref[...] = p[:, 3 * _D:4 * _D]


_node_mm = pl.pallas_call(
    _node_mm_body,
    grid=(_N // _BN,),
    in_specs=[
        pl.BlockSpec((_BN, _D), lambda i: (i, 0)),
        pl.BlockSpec((_D, 4 * _D), lambda i: (0, 0)),
    ],
    out_specs=[pl.BlockSpec((_BN, _D), lambda i: (i, 0))] * 4,
    out_shape=[jax.ShapeDtypeStruct((_N, _D), jnp.float32)] * 4,
)


def _edge1_body(pT_ref, e_ref, g1_ref, ehat_ref, gate_ref, stats_ref):
    e_hat = (jnp.dot(e_ref[...], pT_ref[...], preferred_element_type=jnp.float32)
             + g1_ref[...])
    ehat_ref[...] = e_hat
    gate_ref[...] = jax.nn.sigmoid(e_hat)
    s1 = jnp.sum(e_hat, axis=0, keepdims=True)
    s2 = jnp.sum(e_hat * e_hat, axis=0, keepdims=True)
    blk = jnp.concatenate([s1, s2, jnp.zeros((6, _D), jnp.float32)], axis=0)

    @pl.when(pl.program_id(0) == 0)
    def _():
        stats_ref[...] = blk

    @pl.when(pl.program_id(0) != 0)
    def _():
        stats_ref[...] += blk


_edge_pass1 = pl.pallas_call(
    _edge1_body,
    grid=(_GE,),
    in_specs=[
        pl.BlockSpec((_D, _D), lambda i: (0, 0)),
        pl.BlockSpec((_BE, _D), lambda i: (i, 0)),
        pl.BlockSpec((_BE, _D), lambda i: (i, 0)),
    ],
    out_specs=[
        pl.BlockSpec((_BE, _D), lambda i: (i, 0)),
        pl.BlockSpec((_BE, _D), lambda i: (i, 0)),
        pl.BlockSpec((8, _D), lambda i: (0, 0)),
    ],
    out_shape=[
        jax.ShapeDtypeStruct((_E, _D), jnp.float32),
        jax.ShapeDtypeStruct((_E, _D), jnp.float32),
        jax.ShapeDtypeStruct((8, _D), jnp.float32),
    ],
)


def _edge2_body(stats_ref, m1T_ref, m2T_ref, b1_ref, b2_ref, g_ref, bta_ref,
                e_ref, ehat_ref, enew_ref):
    mean = stats_ref[0:1, :] * (1.0 / _E)
    var = stats_ref[1:2, :] * (1.0 / _E) - mean * mean
    rstd = lax.rsqrt(var + 1e-5)
    x = (ehat_ref[...] - mean) * rstd * g_ref[...] + bta_ref[...]
    t = jnp.maximum(
        jnp.dot(x, m1T_ref[...], preferred_element_type=jnp.float32)
        + b1_ref[...], 0.0)
    y = jnp.dot(t, m2T_ref[...], preferred_element_type=jnp.float32) + b2_ref[...]
    enew_ref[...] = e_ref[...] + y


_edge_pass2 = pl.pallas_call(
    _edge2_body,
    grid=(_GE,),
    in_specs=[
        pl.BlockSpec((8, _D), lambda i: (0, 0)),
        pl.BlockSpec((_D, _D), lambda i: (0, 0)),
        pl.BlockSpec((_D, _D), lambda i: (0, 0)),
        pl.BlockSpec((1, _D), lambda i: (0, 0)),
        pl.BlockSpec((1, _D), lambda i: (0, 0)),
        pl.BlockSpec((1, _D), lambda i: (0, 0)),
        pl.BlockSpec((1, _D), lambda i: (0, 0)),
        pl.BlockSpec((_BE, _D), lambda i: (i, 0)),
        pl.BlockSpec((_BE, _D), lambda i: (i, 0)),
    ],
    out_specs=pl.BlockSpec((_BE, _D), lambda i: (i, 0)),
    out_shape=jax.ShapeDtypeStruct((_E, _D), jnp.float32),
)


def _node_bn_body(h_ref, uh_ref, a0_ref, a1_ref, g_ref, bta_ref, hnew_ref):
    z = uh_ref[...] + a0_ref[...] + a1_ref[...]
    mu = jnp.mean(z, axis=0, keepdims=True)
    var = jnp.mean((z - mu) * (z - mu), axis=0, keepdims=True)
    zn = (z - mu) * lax.rsqrt(var + 1e-5) * g_ref[...] + bta_ref[...]
    hnew_ref[...] = h_ref[...] + zn


_node_pass = pl.pallas_call(
    _node_bn_body,
    out_shape=jax.ShapeDtypeStruct((_N, _D), jnp.float32),
)


@functools.lru_cache(maxsize=1)
def _make_gather():
    @functools.partial(
        pl.kernel,
        mesh=_sc_mesh(),
        out_type=jax.ShapeDtypeStruct((_E, _D), jnp.float32),
        scratch_types=[
            pltpu.VMEM((_NCHUNK, _CH), jnp.int32),
            pltpu.VMEM((_NCHUNK, _CH), jnp.int32),
            pltpu.VMEM((_CH, _D), jnp.float32),
            pltpu.VMEM((_CH, _D), jnp.float32),
            pltpu.VMEM((_CH, _D), jnp.float32),
            pltpu.VMEM((_CH, _D), jnp.float32),
            pltpu.SemaphoreType.DMA,
            pltpu.SemaphoreType.DMA,
            pltpu.SemaphoreType.DMA,
            pltpu.SemaphoreType.DMA,
            pltpu.SemaphoreType.DMA,
            pltpu.SemaphoreType.DMA,
        ],
    )
    def _gather(qh, rh, src3, dst3, g1,
                src_i, dst_i, qb0, qb1, rb0, rb1,
                gq0, gq1, gr0, gr1, os0, os1):
        wid = lax.axis_index("s") * _NC + lax.axis_index("c")
        base0 = wid * _EPW
        qbs, rbs = (qb0, qb1), (rb0, rb1)
        gqs, grs, oss = (gq0, gq1), (gr0, gr1), (os0, os1)
        pltpu.sync_copy(src3.at[wid], src_i)
        pltpu.sync_copy(dst3.at[wid], dst_i)
        pltpu.async_copy(qh.at[src_i.at[0]], qbs[0], gqs[0])
        pltpu.async_copy(rh.at[dst_i.at[0]], rbs[0], grs[0])

        @pl.loop(0, _NCHUNK + 1, step=2)
        def pair(ii):
            for b in range(2):
                i = ii + b

                @pl.when(i < _NCHUNK)
                def _():
                    nb = 1 - b
                    @pl.when(i > 0)
                    def _():
                        pltpu.make_async_copy(
                            qbs[nb], g1.at[pl.ds(base0, _CH)], oss[nb]).wait()

                    @pl.when(i + 1 < _NCHUNK)
                    def _():
                        pltpu.async_copy(qh.at[src_i.at[i + 1]], qbs[nb],
                                         gqs[nb])
                        pltpu.async_copy(rh.at[dst_i.at[i + 1]], rbs[nb],
                                         grs[nb])

                    pltpu.make_async_copy(qh.at[src_i.at[i]], qbs[b],
                                          gqs[b]).wait()
                    pltpu.make_async_copy(rh.at[dst_i.at[i]], rbs[b],
                                          grs[b]).wait()

                    @pl.loop(0, _CH, unroll=4)
                    def row(r):
                        for g in range(_D // 16):
                            sl = pl.ds(g * 16, 16)
                            plsc.addupdate(qbs[b].at[r, sl], rbs[b][r, sl])

                    pltpu.async_copy(
                        qbs[b], g1.at[pl.ds(base0 + i * _CH, _CH)], oss[b])

        pltpu.make_async_copy(qbs[0], g1.at[pl.ds(base0, _CH)], oss[0]).wait()

    return _gather


@functools.lru_cache(maxsize=1)
def _make_scatter():
    @functools.partial(
        pl.kernel,
        mesh=_sc_mesh(),
        out_type=jax.ShapeDtypeStruct((_NC, _NP, _D), jnp.float32),
        scratch_types=[
            pltpu.VMEM((_CH,), jnp.int32),
            pltpu.VMEM((_CH,), jnp.int32),
            pltpu.VMEM((_CH, _D), jnp.float32),
            pltpu.VMEM((_CH, _D), jnp.float32),
            pltpu.VMEM_SHARED((_NP, _D), jnp.float32),
            pltpu.SemaphoreType.DMA,
            pltpu.SemaphoreType.DMA,
        ],
    )
    def _scatter(gate, vh, srcf, dstf, zeros_hbm, out,
                 sv, dv, gb, vb, agg_sh, gsem, ssem):
        c = lax.axis_index("c")
        s = lax.axis_index("s")
        wid = s * _NC + c
        pltpu.sync_copy(zeros_hbm.at[pl.ds(s * _RPT, _RPT)],
                        agg_sh.at[pl.ds(s * _RPT, _RPT)])
        plsc.subcore_barrier()
        base0 = wid * _EPW

        def body(i, carry):
            base = base0 + i * _CH
            pltpu.sync_copy(dstf.at[pl.ds(base, _CH)], dv)
            cv = pltpu.async_copy(vh.at[dv], vb, gsem)
            pltpu.sync_copy(srcf.at[pl.ds(base, _CH)], sv)
            pltpu.sync_copy(gate.at[pl.ds(base, _CH)], gb)
            cv.wait()

            @pl.loop(0, _CH, unroll=4)
            def row(r):
                for g in range(_D // 16):
                    sl = pl.ds(g * 16, 16)
                    vb[r, sl] = gb[r, sl] * vb[r, sl]

            pltpu.sync_copy(vb, agg_sh.at[sv], add=True)
            return carry

        lax.fori_loop(0, _NCHUNK, body, 0)
        plsc.subcore_barrier()
        pltpu.sync_copy(agg_sh.at[pl.ds(s * _RPT, _RPT)],
                        out.at[c, pl.ds(s * _RPT, _RPT)])

    return _scatter


def kernel(h, e, edge_index, P_w, Q_w, R_w, U_w, V_w,
           m1_w, m1_b, m2_w, m2_b, e_gamma, e_beta, n_gamma, n_beta):
    src3 = edge_index[0].reshape(_NW, _NCHUNK, _CH)
    dst3 = edge_index[1].reshape(_NW, _NCHUNK, _CH)
    w4 = jnp.concatenate([Q_w.T, R_w.T, V_w.T, U_w.T], axis=1)
    qh, rh, vh, uh = _node_mm(h, w4)
    g1 = _make_gather()(qh, rh, src3, dst3)
    ehat, gate, stats = _edge_pass1(P_w.T, e, g1)
    aggc = _make_scatter()(gate, vh, edge_index[0], edge_index[1],
                           jnp.zeros((_NP, _D), jnp.float32))
    e_new = _edge_pass2(stats, m1_w.T, m2_w.T,
                        m1_b.reshape(1, _D), m2_b.reshape(1, _D),
                        e_gamma.reshape(1, _D), e_beta.reshape(1, _D),
                        e, ehat)
    h_new = _node_pass(h, uh, aggc[0, :_N], aggc[1, :_N],
                       n_gamma.reshape(1, _D), n_beta.reshape(1, _D))
    return (h_new, e_new)


# speedup vs baseline: 2.2964x; 1.1309x over previous
import functools

import jax
import jax.numpy as jnp
from jax import lax
from jax.experimental import pallas as pl
from jax.experimental.pallas import tpu as pltpu
from jax.experimental.pallas import tpu_sc as plsc

_N, _E, _D = 10000, 320000, 128
_NC, _NS = 2, 16
_NW = _NC * _NS
_EPW = _E // _NW
_CH = 80
_NCHUNK = _EPW // _CH
_NP = 10240
_RPT = _NP // _NS
_CS = 64
_NCS = _E // _CS
_BE = 4000
_GE = _E // _BE
_BN = 2000

@functools.lru_cache(maxsize=1)
def _sc_mesh():
    return plsc.VectorSubcoreMesh(
        core_axis_name="c", subcore_axis_name="s",
        num_cores=_NC, num_subcores=_NS)


def _node_mm_body(h_ref, w_ref, qh_ref, rh_ref, vh_ref, uh_ref):
    p = jnp.dot(h_ref[...], w_ref[...], preferred_element_type=jnp.float32)
    qh_ref[...] = p[:, 0 * _D:1 * _D]
    rh_ref[...] = p[:, 1 * _D:2 * _D]
    vh_ref[...] = p[:, 2 * _D:3 * _D]
    uh_ref[...] = p[:, 3 * _D:4 * _D]


_node_mm = pl.pallas_call(
    _node_mm_body,
    grid=(_N // _BN,),
    in_specs=[
        pl.BlockSpec((_BN, _D), lambda i: (i, 0)),
        pl.BlockSpec((_D, 4 * _D), lambda i: (0, 0)),
    ],
    out_specs=[pl.BlockSpec((_BN, _D), lambda i: (i, 0))] * 4,
    out_shape=[jax.ShapeDtypeStruct((_N, _D), jnp.float32)] * 4,
)


def _edge1_body(pT_ref, e_ref, g1_ref, ehat_ref, gate_ref, stats_ref):
    e_hat = (jnp.dot(e_ref[...], pT_ref[...], preferred_element_type=jnp.float32)
             + g1_ref[...])
    ehat_ref[...] = e_hat
    gate_ref[...] = jax.nn.sigmoid(e_hat)
    s1 = jnp.sum(e_hat, axis=0, keepdims=True)
    s2 = jnp.sum(e_hat * e_hat, axis=0, keepdims=True)
    blk = jnp.concatenate([s1, s2, jnp.zeros((6, _D), jnp.float32)], axis=0)

    @pl.when(pl.program_id(0) == 0)
    def _():
        stats_ref[...] = blk

    @pl.when(pl.program_id(0) != 0)
    def _():
        stats_ref[...] += blk


_edge_pass1 = pl.pallas_call(
    _edge1_body,
    grid=(_GE,),
    in_specs=[
        pl.BlockSpec((_D, _D), lambda i: (0, 0)),
        pl.BlockSpec((_BE, _D), lambda i: (i, 0)),
        pl.BlockSpec((_BE, _D), lambda i: (i, 0)),
    ],
    out_specs=[
        pl.BlockSpec((_BE, _D), lambda i: (i, 0)),
        pl.BlockSpec((_BE, _D), lambda i: (i, 0)),
        pl.BlockSpec((8, _D), lambda i: (0, 0)),
    ],
    out_shape=[
        jax.ShapeDtypeStruct((_E, _D), jnp.float32),
        jax.ShapeDtypeStruct((_E, _D), jnp.float32),
        jax.ShapeDtypeStruct((8, _D), jnp.float32),
    ],
)


def _edge2_body(stats_ref, m1T_ref, m2T_ref, b1_ref, b2_ref, g_ref, bta_ref,
                e_ref, ehat_ref, enew_ref):
    mean = stats_ref[0:1, :] * (1.0 / _E)
    var = stats_ref[1:2, :] * (1.0 / _E) - mean * mean
    rstd = lax.rsqrt(var + 1e-5)
    x = (ehat_ref[...] - mean) * rstd * g_ref[...] + bta_ref[...]
    t = jnp.maximum(
        jnp.dot(x, m1T_ref[...], preferred_element_type=jnp.float32)
        + b1_ref[...], 0.0)
    y = jnp.dot(t, m2T_ref[...], preferred_element_type=jnp.float32) + b2_ref[...]
    enew_ref[...] = e_ref[...] + y


_edge_pass2 = pl.pallas_call(
    _edge2_body,
    grid=(_GE,),
    in_specs=[
        pl.BlockSpec((8, _D), lambda i: (0, 0)),
        pl.BlockSpec((_D, _D), lambda i: (0, 0)),
        pl.BlockSpec((_D, _D), lambda i: (0, 0)),
        pl.BlockSpec((1, _D), lambda i: (0, 0)),
        pl.BlockSpec((1, _D), lambda i: (0, 0)),
        pl.BlockSpec((1, _D), lambda i: (0, 0)),
        pl.BlockSpec((1, _D), lambda i: (0, 0)),
        pl.BlockSpec((_BE, _D), lambda i: (i, 0)),
        pl.BlockSpec((_BE, _D), lambda i: (i, 0)),
    ],
    out_specs=pl.BlockSpec((_BE, _D), lambda i: (i, 0)),
    out_shape=jax.ShapeDtypeStruct((_E, _D), jnp.float32),
)


def _node_bn_body(h_ref, uh_ref, a0_ref, a1_ref, g_ref, bta_ref, hnew_ref):
    z = uh_ref[...] + a0_ref[...] + a1_ref[...]
    mu = jnp.mean(z, axis=0, keepdims=True)
    var = jnp.mean((z - mu) * (z - mu), axis=0, keepdims=True)
    zn = (z - mu) * lax.rsqrt(var + 1e-5) * g_ref[...] + bta_ref[...]
    hnew_ref[...] = h_ref[...] + zn


_node_pass = pl.pallas_call(
    _node_bn_body,
    out_shape=jax.ShapeDtypeStruct((_N, _D), jnp.float32),
)


@functools.lru_cache(maxsize=1)
def _make_gather():
    @functools.partial(
        pl.kernel,
        mesh=_sc_mesh(),
        out_type=jax.ShapeDtypeStruct((_E, _D), jnp.float32),
        scratch_types=[
            pltpu.VMEM((_NCHUNK, _CH), jnp.int32),
            pltpu.VMEM((_NCHUNK, _CH), jnp.int32),
            pltpu.VMEM((_CH, _D), jnp.float32),
            pltpu.VMEM((_CH, _D), jnp.float32),
            pltpu.VMEM((_CH, _D), jnp.float32),
            pltpu.VMEM((_CH, _D), jnp.float32),
            pltpu.SemaphoreType.DMA,
            pltpu.SemaphoreType.DMA,
            pltpu.SemaphoreType.DMA,
            pltpu.SemaphoreType.DMA,
            pltpu.SemaphoreType.DMA,
            pltpu.SemaphoreType.DMA,
        ],
    )
    def _gather(qh, rh, src3, dst3, g1,
                src_i, dst_i, qb0, qb1, rb0, rb1,
                gq0, gq1, gr0, gr1, os0, os1):
        wid = lax.axis_index("s") * _NC + lax.axis_index("c")
        base0 = wid * _EPW
        qbs, rbs = (qb0, qb1), (rb0, rb1)
        gqs, grs, oss = (gq0, gq1), (gr0, gr1), (os0, os1)
        pltpu.sync_copy(src3.at[wid], src_i)
        pltpu.sync_copy(dst3.at[wid], dst_i)
        pltpu.async_copy(qh.at[src_i.at[0]], qbs[0], gqs[0])
        pltpu.async_copy(rh.at[dst_i.at[0]], rbs[0], grs[0])

        @pl.loop(0, _NCHUNK + 1, step=2)
        def pair(ii):
            for b in range(2):
                i = ii + b

                @pl.when(i < _NCHUNK)
                def _():
                    nb = 1 - b
                    @pl.when(i > 0)
                    def _():
                        pltpu.make_async_copy(
                            qbs[nb], g1.at[pl.ds(base0, _CH)], oss[nb]).wait()

                    @pl.when(i + 1 < _NCHUNK)
                    def _():
                        pltpu.async_copy(qh.at[src_i.at[i + 1]], qbs[nb],
                                         gqs[nb])
                        pltpu.async_copy(rh.at[dst_i.at[i + 1]], rbs[nb],
                                         grs[nb])

                    pltpu.make_async_copy(qh.at[src_i.at[i]], qbs[b],
                                          gqs[b]).wait()
                    pltpu.make_async_copy(rh.at[dst_i.at[i]], rbs[b],
                                          grs[b]).wait()

                    @pl.loop(0, _CH, unroll=4)
                    def row(r):
                        for g in range(_D // 16):
                            sl = pl.ds(g * 16, 16)
                            plsc.addupdate(qbs[b].at[r, sl], rbs[b][r, sl])

                    pltpu.async_copy(
                        qbs[b], g1.at[pl.ds(base0 + i * _CH, _CH)], oss[b])

        pltpu.make_async_copy(qbs[0], g1.at[pl.ds(base0, _CH)], oss[0]).wait()

    return _gather


@functools.lru_cache(maxsize=1)
def _make_scatter():
    @functools.partial(
        pl.kernel,
        mesh=_sc_mesh(),
        out_type=jax.ShapeDtypeStruct((_NC, _NP, _D), jnp.float32),
        scratch_types=[
            pltpu.VMEM((_CS,), jnp.int32),
            pltpu.VMEM((_CS,), jnp.int32),
            pltpu.VMEM((_CS,), jnp.int32),
            pltpu.VMEM((_CS,), jnp.int32),
            pltpu.VMEM((_CS, _D), jnp.float32),
            pltpu.VMEM((_CS, _D), jnp.float32),
            pltpu.VMEM((_CS, _D), jnp.float32),
            pltpu.VMEM_SHARED((_NP, _D), jnp.float32),
            pltpu.SemaphoreType.DMA,
            pltpu.SemaphoreType.DMA,
            pltpu.SemaphoreType.DMA,
            pltpu.SemaphoreType.DMA,
            pltpu.SemaphoreType.DMA,
            pltpu.SemaphoreType.DMA,
            pltpu.SemaphoreType.DMA,
        ],
    )
    def _scatter(gate, vh, srcf, dstf, zeros_hbm, out,
                 sv0, sv1, dv0, dv1, vb0, vb1, gb, agg_sh,
                 is0, is1, gv0, gv1, ss0, ss1, gbsem):
        c = lax.axis_index("c")
        s = lax.axis_index("s")
        wid = s * _NC + c
        svs, dvs, vbs = (sv0, sv1), (dv0, dv1), (vb0, vb1)
        iss, gvs, sss = (is0, is1), (gv0, gv1), (ss0, ss1)
        pltpu.sync_copy(zeros_hbm.at[pl.ds(s * _RPT, _RPT)],
                        agg_sh.at[pl.ds(s * _RPT, _RPT)])
        cnt = jnp.where(wid < 8, 157, 156)
        base0 = (wid * 156 + jnp.minimum(wid, 8)) * _CS
        pltpu.sync_copy(srcf.at[pl.ds(base0, _CS)], svs[0])
        pltpu.sync_copy(dstf.at[pl.ds(base0, _CS)], dvs[0])
        pltpu.async_copy(vh.at[dvs[0]], vbs[0], gvs[0])
        pltpu.sync_copy(gate.at[pl.ds(base0, _CS)], gb)
        plsc.subcore_barrier()

        @pl.loop(0, 158, step=2)
        def pair(ii):
            for b in range(2):
                i = ii + b

                @pl.when(i < cnt)
                def _():
                    nb = 1 - b
                    base = base0 + i * _CS

                    @pl.when(i > 0)
                    def _():
                        pltpu.make_async_copy(
                            gate.at[pl.ds(base0, _CS)], vbs[nb],
                            sss[nb]).wait()

                    @pl.when(i + 1 < cnt)
                    def _():
                        pltpu.async_copy(
                            srcf.at[pl.ds(base + _CS, _CS)], svs[nb], iss[nb])
                        pltpu.async_copy(
                            dstf.at[pl.ds(base + _CS, _CS)], dvs[nb], iss[nb])

                    pltpu.make_async_copy(
                        gate.at[pl.ds(base0, _CS)], vbs[b], gvs[b]).wait()

                    @pl.when(i > 0)
                    def _():
                        pltpu.make_async_copy(
                            gate.at[pl.ds(base0, _CS)], gb, gbsem).wait()

                    @pl.when(i + 1 < cnt)
                    def _():
                        pltpu.make_async_copy(
                            srcf.at[pl.ds(base0, _CS)], svs[nb],
                            iss[nb]).wait()
                        pltpu.make_async_copy(
                            dstf.at[pl.ds(base0, _CS)], dvs[nb],
                            iss[nb]).wait()
                        pltpu.async_copy(vh.at[dvs[nb]], vbs[nb], gvs[nb])

                    @pl.loop(0, _CS, unroll=4)
                    def row(r):
                        for g in range(_D // 16):
                            sl = pl.ds(g * 16, 16)
                            vbs[b][r, sl] = gb[r, sl] * vbs[b][r, sl]

                    @pl.when(i + 1 < cnt)
                    def _():
                        pltpu.async_copy(gate.at[pl.ds(base + _CS, _CS)],
                                         gb, gbsem)

                    pltpu.async_copy(vbs[b], agg_sh.at[svs[b]], sss[b],
                                     add=True)

        @pl.when(cnt == 157)
        def _():
            pltpu.make_async_copy(
                gate.at[pl.ds(base0, _CS)], vbs[0], sss[0]).wait()

        @pl.when(cnt == 156)
        def _():
            pltpu.make_async_copy(
                gate.at[pl.ds(base0, _CS)], vbs[1], sss[1]).wait()

        plsc.subcore_barrier()
        pltpu.sync_copy(agg_sh.at[pl.ds(s * _RPT, _RPT)],
                        out.at[c, pl.ds(s * _RPT, _RPT)])

    return _scatter


def kernel(h, e, edge_index, P_w, Q_w, R_w, U_w, V_w,
           m1_w, m1_b, m2_w, m2_b, e_gamma, e_beta, n_gamma, n_beta):
    src3 = edge_index[0].reshape(_NW, _NCHUNK, _CH)
    dst3 = edge_index[1].reshape(_NW, _NCHUNK, _CH)
    w4 = jnp.concatenate([Q_w.T, R_w.T, V_w.T, U_w.T], axis=1)
    qh, rh, vh, uh = _node_mm(h, w4)
    g1 = _make_gather()(qh, rh, src3, dst3)
    ehat, gate, stats = _edge_pass1(P_w.T, e, g1)
    aggc = _make_scatter()(gate, vh, edge_index[0], edge_index[1],
                           jnp.zeros((_NP, _D), jnp.float32))
    e_new = _edge_pass2(stats, m1_w.T, m2_w.T,
                        m1_b.reshape(1, _D), m2_b.reshape(1, _D),
                        e_gamma.reshape(1, _D), e_beta.reshape(1, _D),
                        e, ehat)
    h_new = _node_pass(h, uh, aggc[0, :_N], aggc[1, :_N],
                       n_gamma.reshape(1, _D), n_beta.reshape(1, _D))
    return (h_new, e_new)


# speedup vs baseline: 2.3274x; 1.0135x over previous
import functools

import jax
import jax.numpy as jnp
from jax import lax
from jax.experimental import pallas as pl
from jax.experimental.pallas import tpu as pltpu
from jax.experimental.pallas import tpu_sc as plsc

_N, _E, _D = 10000, 320000, 128
_EH = _E // 2
_NC, _NS = 2, 16
_NW = _NC * _NS
_CH = 80
_GCMAX = 63
_NP = 10240
_RPT = _NP // _NS
_CS = 64
_SCMAX = 79
_BE = 4000
_GEH = _EH // _BE
_BN = 2000


@functools.lru_cache(maxsize=1)
def _sc_mesh():
    return plsc.VectorSubcoreMesh(
        core_axis_name="c", subcore_axis_name="s",
        num_cores=_NC, num_subcores=_NS)


def _node_mm_body(h_ref, w_ref, qh_ref, rh_ref, vh_ref, uh_ref):
    p = jnp.dot(h_ref[...], w_ref[...], preferred_element_type=jnp.float32)
    qh_ref[...] = p[:, 0 * _D:1 * _D]
    rh_ref[...] = p[:, 1 * _D:2 * _D]
    vh_ref[...] = p[:, 2 * _D:3 * _D]
    uh_ref[...] = p[:, 3 * _D:4 * _D]


_node_mm = pl.pallas_call(
    _node_mm_body,
    grid=(_N // _BN,),
    in_specs=[
        pl.BlockSpec((_BN, _D), lambda i: (i, 0)),
        pl.BlockSpec((_D, 4 * _D), lambda i: (0, 0)),
    ],
    out_specs=[pl.BlockSpec((_BN, _D), lambda i: (i, 0))] * 4,
    out_shape=[jax.ShapeDtypeStruct((_N, _D), jnp.float32)] * 4,
)


def _edge1_body(pT_ref, e_ref, g1_ref, ehat_ref, gate_ref, stats_ref):
    e_hat = (jnp.dot(e_ref[...], pT_ref[...], preferred_element_type=jnp.float32)
             + g1_ref[...])
    ehat_ref[...] = e_hat
    gate_ref[...] = jax.nn.sigmoid(e_hat)
    s1 = jnp.sum(e_hat, axis=0, keepdims=True)
    s2 = jnp.sum(e_hat * e_hat, axis=0, keepdims=True)
    blk = jnp.concatenate([s1, s2, jnp.zeros((6, _D), jnp.float32)], axis=0)

    @pl.when(pl.program_id(0) == 0)
    def _():
        stats_ref[...] = blk

    @pl.when(pl.program_id(0) != 0)
    def _():
        stats_ref[...] += blk


@functools.lru_cache(maxsize=2)
def _make_edge_pass1(h):
    return pl.pallas_call(
        _edge1_body,
        grid=(_GEH,),
        in_specs=[
            pl.BlockSpec((_D, _D), lambda i: (0, 0)),
            pl.BlockSpec((_BE, _D), lambda i, h=h: (i + h * _GEH, 0)),
            pl.BlockSpec((_BE, _D), lambda i: (i, 0)),
        ],
        out_specs=[
            pl.BlockSpec((_BE, _D), lambda i: (i, 0)),
            pl.BlockSpec((_BE, _D), lambda i: (i, 0)),
            pl.BlockSpec((8, _D), lambda i: (0, 0)),
        ],
        out_shape=[
            jax.ShapeDtypeStruct((_EH, _D), jnp.float32),
            jax.ShapeDtypeStruct((_EH, _D), jnp.float32),
            jax.ShapeDtypeStruct((8, _D), jnp.float32),
        ],
    )


def _edge2_body(buf_ref, sa_ref, sb_ref, m1T_ref, m2T_ref, b1_ref, b2_ref,
                g_ref, bta_ref, e_ref, ehat_ref, enew_ref):
    mean = (sa_ref[0:1, :] + sb_ref[0:1, :]) * (1.0 / _E)
    var = (sa_ref[1:2, :] + sb_ref[1:2, :]) * (1.0 / _E) - mean * mean
    rstd = lax.rsqrt(var + 1e-5)
    x = (ehat_ref[...] - mean) * rstd * g_ref[...] + bta_ref[...]
    t = jnp.maximum(
        jnp.dot(x, m1T_ref[...], preferred_element_type=jnp.float32)
        + b1_ref[...], 0.0)
    y = jnp.dot(t, m2T_ref[...], preferred_element_type=jnp.float32) + b2_ref[...]
    enew_ref[...] = e_ref[...] + y


@functools.lru_cache(maxsize=2)
def _make_edge_pass2(h):
    return pl.pallas_call(
        _edge2_body,
        grid=(_GEH,),
        in_specs=[
            pl.BlockSpec((_BE, _D), lambda i, h=h: (i + h * _GEH, 0)),
            pl.BlockSpec((8, _D), lambda i: (0, 0)),
            pl.BlockSpec((8, _D), lambda i: (0, 0)),
            pl.BlockSpec((_D, _D), lambda i: (0, 0)),
            pl.BlockSpec((_D, _D), lambda i: (0, 0)),
            pl.BlockSpec((1, _D), lambda i: (0, 0)),
            pl.BlockSpec((1, _D), lambda i: (0, 0)),
            pl.BlockSpec((1, _D), lambda i: (0, 0)),
            pl.BlockSpec((1, _D), lambda i: (0, 0)),
            pl.BlockSpec((_BE, _D), lambda i, h=h: (i + h * _GEH, 0)),
            pl.BlockSpec((_BE, _D), lambda i: (i, 0)),
        ],
        out_specs=pl.BlockSpec((_BE, _D), lambda i, h=h: (i + h * _GEH, 0)),
        out_shape=jax.ShapeDtypeStruct((_E, _D), jnp.float32),
        input_output_aliases={0: 0},
    )


def _node_bn_body(h_ref, uh_ref, a0_ref, a1_ref, b0_ref, b1_ref,
                  g_ref, bta_ref, hnew_ref):
    z = (uh_ref[...] + a0_ref[...] + a1_ref[...]
         + b0_ref[...] + b1_ref[...])
    mu = jnp.mean(z, axis=0, keepdims=True)
    var = jnp.mean((z - mu) * (z - mu), axis=0, keepdims=True)
    zn = (z - mu) * lax.rsqrt(var + 1e-5) * g_ref[...] + bta_ref[...]
    hnew_ref[...] = h_ref[...] + zn


_node_pass = pl.pallas_call(
    _node_bn_body,
    out_shape=jax.ShapeDtypeStruct((_N, _D), jnp.float32),
)


@functools.lru_cache(maxsize=2)
def _make_gather(h):
    @functools.partial(
        pl.kernel,
        mesh=_sc_mesh(),
        out_type=jax.ShapeDtypeStruct((_EH, _D), jnp.float32),
        scratch_types=[
            pltpu.VMEM((_GCMAX, 1, _CH), jnp.int32),
            pltpu.VMEM((_GCMAX, 1, _CH), jnp.int32),
            pltpu.VMEM((_CH, _D), jnp.float32),
            pltpu.VMEM((_CH, _D), jnp.float32),
            pltpu.VMEM((_CH, _D), jnp.float32),
            pltpu.VMEM((_CH, _D), jnp.float32),
            pltpu.SemaphoreType.DMA,
            pltpu.SemaphoreType.DMA,
            pltpu.SemaphoreType.DMA,
            pltpu.SemaphoreType.DMA,
            pltpu.SemaphoreType.DMA,
            pltpu.SemaphoreType.DMA,
        ],
    )
    def _gather(qh, rh, src3, dst3, g1,
                src_i, dst_i, qb0, qb1, rb0, rb1,
                gq0, gq1, gr0, gr1, os0, os1):
        wid = lax.axis_index("s") * _NC + lax.axis_index("c")
        cnt = jnp.where(wid < 16, 63, 62)
        basec = wid * 62 + jnp.minimum(wid, 16)
        base0 = basec * _CH
        qbs, rbs = (qb0, qb1), (rb0, rb1)
        gqs, grs, oss = (gq0, gq1), (gr0, gr1), (os0, os1)
        pltpu.sync_copy(src3.at[pl.ds(h * 2000 + basec, _GCMAX)], src_i)
        pltpu.sync_copy(dst3.at[pl.ds(h * 2000 + basec, _GCMAX)], dst_i)
        pltpu.async_copy(qh.at[src_i.at[0, 0]], qbs[0], gqs[0])
        pltpu.async_copy(rh.at[dst_i.at[0, 0]], rbs[0], grs[0])

        @pl.loop(0, _GCMAX + 1, step=2)
        def pair(ii):
            for b in range(2):
                i = ii + b

                @pl.when(i < cnt)
                def _():
                    nb = 1 - b
                    @pl.when(i > 0)
                    def _():
                        pltpu.make_async_copy(
                            qbs[nb], g1.at[pl.ds(base0, _CH)], oss[nb]).wait()

                    @pl.when(i + 1 < cnt)
                    def _():
                        pltpu.async_copy(qh.at[src_i.at[i + 1, 0]], qbs[nb],
                                         gqs[nb])
                        pltpu.async_copy(rh.at[dst_i.at[i + 1, 0]], rbs[nb],
                                         grs[nb])

                    pltpu.make_async_copy(qh.at[src_i.at[i, 0]], qbs[b],
                                          gqs[b]).wait()
                    pltpu.make_async_copy(rh.at[dst_i.at[i, 0]], rbs[b],
                                          grs[b]).wait()

                    @pl.loop(0, _CH, unroll=4)
                    def row(r):
                        for g in range(_D // 16):
                            sl = pl.ds(g * 16, 16)
                            plsc.addupdate(qbs[b].at[r, sl], rbs[b][r, sl])

                    pltpu.async_copy(
                        qbs[b], g1.at[pl.ds(base0 + i * _CH, _CH)], oss[b])

        @pl.when(cnt == 63)
        def _():
            pltpu.make_async_copy(qbs[0], g1.at[pl.ds(base0, _CH)],
                                  oss[0]).wait()

        @pl.when(cnt == 62)
        def _():
            pltpu.make_async_copy(qbs[1], g1.at[pl.ds(base0, _CH)],
                                  oss[1]).wait()

    return _gather


@functools.lru_cache(maxsize=1)
def _make_scatter():
    @functools.partial(
        pl.kernel,
        mesh=_sc_mesh(),
        out_type=jax.ShapeDtypeStruct((_NC, _NP, _D), jnp.float32),
        scratch_types=[
            pltpu.VMEM((_CS,), jnp.int32),
            pltpu.VMEM((_CS,), jnp.int32),
            pltpu.VMEM((_CS,), jnp.int32),
            pltpu.VMEM((_CS,), jnp.int32),
            pltpu.VMEM((_CS, _D), jnp.float32),
            pltpu.VMEM((_CS, _D), jnp.float32),
            pltpu.VMEM((_CS, _D), jnp.float32),
            pltpu.VMEM_SHARED((_NP, _D), jnp.float32),
            pltpu.SemaphoreType.DMA,
            pltpu.SemaphoreType.DMA,
            pltpu.SemaphoreType.DMA,
            pltpu.SemaphoreType.DMA,
            pltpu.SemaphoreType.DMA,
            pltpu.SemaphoreType.DMA,
            pltpu.SemaphoreType.DMA,
        ],
    )
    def _scatter(gate, vh, srcf, dstf, zeros_hbm, out,
                 sv0, sv1, dv0, dv1, vb0, vb1, gb, agg_sh,
                 is0, is1, gv0, gv1, ss0, ss1, gbsem):
        c = lax.axis_index("c")
        s = lax.axis_index("s")
        wid = s * _NC + c
        svs, dvs, vbs = (sv0, sv1), (dv0, dv1), (vb0, vb1)
        iss, gvs, sss = (is0, is1), (gv0, gv1), (ss0, ss1)
        pltpu.sync_copy(zeros_hbm.at[pl.ds(s * _RPT, _RPT)],
                        agg_sh.at[pl.ds(s * _RPT, _RPT)])
        cnt = jnp.where(wid < 4, 79, 78)
        base0 = (wid * 78 + jnp.minimum(wid, 4)) * _CS
        pltpu.sync_copy(srcf.at[pl.ds(base0, _CS)], svs[0])
        pltpu.sync_copy(dstf.at[pl.ds(base0, _CS)], dvs[0])
        pltpu.async_copy(vh.at[dvs[0]], vbs[0], gvs[0])
        pltpu.sync_copy(gate.at[pl.ds(base0, _CS)], gb)
        plsc.subcore_barrier()

        @pl.loop(0, _SCMAX + 1, step=2)
        def pair(ii):
            for b in range(2):
                i = ii + b

                @pl.when(i < cnt)
                def _():
                    nb = 1 - b
                    base = base0 + i * _CS

                    @pl.when(i > 0)
                    def _():
                        pltpu.make_async_copy(
                            gate.at[pl.ds(base0, _CS)], vbs[nb],
                            sss[nb]).wait()

                    @pl.when(i + 1 < cnt)
                    def _():
                        pltpu.async_copy(
                            srcf.at[pl.ds(base + _CS, _CS)], svs[nb], iss[nb])
                        pltpu.async_copy(
                            dstf.at[pl.ds(base + _CS, _CS)], dvs[nb], iss[nb])

                    pltpu.make_async_copy(
                        gate.at[pl.ds(base0, _CS)], vbs[b], gvs[b]).wait()

                    @pl.when(i > 0)
                    def _():
                        pltpu.make_async_copy(
                            gate.at[pl.ds(base0, _CS)], gb, gbsem).wait()

                    @pl.when(i + 1 < cnt)
                    def _():
                        pltpu.make_async_copy(
                            srcf.at[pl.ds(base0, _CS)], svs[nb],
                            iss[nb]).wait()
                        pltpu.make_async_copy(
                            dstf.at[pl.ds(base0, _CS)], dvs[nb],
                            iss[nb]).wait()
                        pltpu.async_copy(vh.at[dvs[nb]], vbs[nb], gvs[nb])

                    @pl.loop(0, _CS, unroll=4)
                    def row(r):
                        for g in range(_D // 16):
                            sl = pl.ds(g * 16, 16)
                            vbs[b][r, sl] = gb[r, sl] * vbs[b][r, sl]

                    @pl.when(i + 1 < cnt)
                    def _():
                        pltpu.async_copy(gate.at[pl.ds(base + _CS, _CS)],
                                         gb, gbsem)

                    pltpu.async_copy(vbs[b], agg_sh.at[svs[b]], sss[b],
                                     add=True)

        @pl.when(cnt == 79)
        def _():
            pltpu.make_async_copy(
                gate.at[pl.ds(base0, _CS)], vbs[0], sss[0]).wait()

        @pl.when(cnt == 78)
        def _():
            pltpu.make_async_copy(
                gate.at[pl.ds(base0, _CS)], vbs[1], sss[1]).wait()

        plsc.subcore_barrier()
        pltpu.sync_copy(agg_sh.at[pl.ds(s * _RPT, _RPT)],
                        out.at[c, pl.ds(s * _RPT, _RPT)])

    return _scatter


def kernel(h, e, edge_index, P_w, Q_w, R_w, U_w, V_w,
           m1_w, m1_b, m2_w, m2_b, e_gamma, e_beta, n_gamma, n_beta):
    src = edge_index[0]
    dst = edge_index[1]
    src3 = jnp.pad(src, (0, 8 * _CH)).reshape(_E // _CH + 8, 1, _CH)
    dst3 = jnp.pad(dst, (0, 8 * _CH)).reshape(_E // _CH + 8, 1, _CH)
    w4 = jnp.concatenate([Q_w.T, R_w.T, V_w.T, U_w.T], axis=1)
    qh, rh, vh, uh = _node_mm(h, w4)
    zagg = jnp.zeros((_NP, _D), jnp.float32)
    ebuf = jnp.zeros((_E, _D), jnp.float32)

    g1a = _make_gather(0)(qh, rh, src3, dst3)
    g1b = _make_gather(1)(qh, rh, src3, dst3)
    ehat_a, gate_a, stats_a = _make_edge_pass1(0)(P_w.T, e, g1a)
    ehat_b, gate_b, stats_b = _make_edge_pass1(1)(P_w.T, e, g1b)
    scat = _make_scatter()
    agg_a = scat(gate_a, vh, src[:_EH], dst[:_EH], zagg)
    agg_b = scat(gate_b, vh, src[_EH:], dst[_EH:], zagg)

    mlp_args = (m1_w.T, m2_w.T, m1_b.reshape(1, _D), m2_b.reshape(1, _D),
                e_gamma.reshape(1, _D), e_beta.reshape(1, _D))
    en0 = _make_edge_pass2(0)(ebuf, stats_a, stats_b, *mlp_args, e, ehat_a)
    e_new = _make_edge_pass2(1)(en0, stats_a, stats_b, *mlp_args, e, ehat_b)
    h_new = _node_pass(h, uh, agg_a[0, :_N], agg_a[1, :_N],
                       agg_b[0, :_N], agg_b[1, :_N],
                       n_gamma.reshape(1, _D), n_beta.reshape(1, _D))
    return (h_new, e_new)


# speedup vs baseline: 2.3671x; 1.0170x over previous
import functools

import jax
import jax.numpy as jnp
from jax import lax
from jax.experimental import pallas as pl
from jax.experimental.pallas import tpu as pltpu
from jax.experimental.pallas import tpu_sc as plsc

_N, _E, _D = 10000, 320000, 128
_EH = _E // 2
_NC, _NS = 2, 16
_NW = _NC * _NS
_CH = 80
_GCMAX = 63
_NP = 10240
_RPT = _NP // _NS
_CS = 64
_SCMAX = 79
_BE = 4000
_GEH = _EH // _BE
_BN = 2000


@functools.lru_cache(maxsize=1)
def _sc_mesh():
    return plsc.VectorSubcoreMesh(
        core_axis_name="c", subcore_axis_name="s",
        num_cores=_NC, num_subcores=_NS)


def _node_mm_body(h_ref, w_ref, qh_ref, rh_ref, vh_ref, uh_ref):
    p = jnp.dot(h_ref[...], w_ref[...], preferred_element_type=jnp.float32)
    qh_ref[...] = p[:, 0 * _D:1 * _D]
    rh_ref[...] = p[:, 1 * _D:2 * _D]
    vh_ref[...] = p[:, 2 * _D:3 * _D]
    uh_ref[...] = p[:, 3 * _D:4 * _D]


_node_mm = pl.pallas_call(
    _node_mm_body,
    grid=(_N // _BN,),
    in_specs=[
        pl.BlockSpec((_BN, _D), lambda i: (i, 0)),
        pl.BlockSpec((_D, 4 * _D), lambda i: (0, 0)),
    ],
    out_specs=[pl.BlockSpec((_BN, _D), lambda i: (i, 0))] * 4,
    out_shape=[jax.ShapeDtypeStruct((_N, _D), jnp.float32)] * 4,
)


def _edge1_body(pT_ref, e_ref, g1_ref, ehat_ref, gate_ref, stats_ref):
    e_hat = (jnp.dot(e_ref[...], pT_ref[...], preferred_element_type=jnp.float32)
             + g1_ref[...])
    ehat_ref[...] = e_hat
    gate_ref[...] = jax.nn.sigmoid(e_hat)
    s1 = jnp.sum(e_hat, axis=0, keepdims=True)
    s2 = jnp.sum(e_hat * e_hat, axis=0, keepdims=True)
    blk = jnp.concatenate([s1, s2, jnp.zeros((6, _D), jnp.float32)], axis=0)

    @pl.when(pl.program_id(0) == 0)
    def _():
        stats_ref[...] = blk

    @pl.when(pl.program_id(0) != 0)
    def _():
        stats_ref[...] += blk


@functools.lru_cache(maxsize=2)
def _make_edge_pass1(h):
    return pl.pallas_call(
        _edge1_body,
        grid=(_GEH,),
        in_specs=[
            pl.BlockSpec((_D, _D), lambda i: (0, 0)),
            pl.BlockSpec((_BE, _D), lambda i, h=h: (i + h * _GEH, 0)),
            pl.BlockSpec((_BE, _D), lambda i: (i, 0)),
        ],
        out_specs=[
            pl.BlockSpec((_BE, _D), lambda i: (i, 0)),
            pl.BlockSpec((_BE, _D), lambda i: (i, 0)),
            pl.BlockSpec((8, _D), lambda i: (0, 0)),
        ],
        out_shape=[
            jax.ShapeDtypeStruct((_EH, _D), jnp.float32),
            jax.ShapeDtypeStruct((_EH, _D), jnp.float32),
            jax.ShapeDtypeStruct((8, _D), jnp.float32),
        ],
    )


def _edge2_body_first(sa_ref, sb_ref, m1T_ref, m2T_ref, b1_ref, b2_ref,
                      g_ref, bta_ref, e_ref, ehat_ref, enew_ref):
    _edge2_body(None, sa_ref, sb_ref, m1T_ref, m2T_ref, b1_ref, b2_ref,
                g_ref, bta_ref, e_ref, ehat_ref, enew_ref)


def _edge2_body(buf_ref, sa_ref, sb_ref, m1T_ref, m2T_ref, b1_ref, b2_ref,
                g_ref, bta_ref, e_ref, ehat_ref, enew_ref):
    mean = (sa_ref[0:1, :] + sb_ref[0:1, :]) * (1.0 / _E)
    var = (sa_ref[1:2, :] + sb_ref[1:2, :]) * (1.0 / _E) - mean * mean
    rstd = lax.rsqrt(var + 1e-5)
    x = (ehat_ref[...] - mean) * rstd * g_ref[...] + bta_ref[...]
    t = jnp.maximum(
        jnp.dot(x, m1T_ref[...], preferred_element_type=jnp.float32)
        + b1_ref[...], 0.0)
    y = jnp.dot(t, m2T_ref[...], preferred_element_type=jnp.float32) + b2_ref[...]
    enew_ref[...] = e_ref[...] + y


@functools.lru_cache(maxsize=2)
def _make_edge_pass2(h):
    buf_spec = ([] if h == 0 else
                [pl.BlockSpec((_BE, _D), lambda i, h=h: (i + h * _GEH, 0))])
    return pl.pallas_call(
        _edge2_body_first if h == 0 else _edge2_body,
        grid=(_GEH,),
        in_specs=buf_spec + [
            pl.BlockSpec((8, _D), lambda i: (0, 0)),
            pl.BlockSpec((8, _D), lambda i: (0, 0)),
            pl.BlockSpec((_D, _D), lambda i: (0, 0)),
            pl.BlockSpec((_D, _D), lambda i: (0, 0)),
            pl.BlockSpec((1, _D), lambda i: (0, 0)),
            pl.BlockSpec((1, _D), lambda i: (0, 0)),
            pl.BlockSpec((1, _D), lambda i: (0, 0)),
            pl.BlockSpec((1, _D), lambda i: (0, 0)),
            pl.BlockSpec((_BE, _D), lambda i, h=h: (i + h * _GEH, 0)),
            pl.BlockSpec((_BE, _D), lambda i: (i, 0)),
        ],
        out_specs=pl.BlockSpec((_BE, _D), lambda i, h=h: (i + h * _GEH, 0)),
        out_shape=jax.ShapeDtypeStruct((_E, _D), jnp.float32),
        input_output_aliases=({} if h == 0 else {0: 0}),
    )


def _node_bn_body(h_ref, uh_ref, a0_ref, a1_ref, b0_ref, b1_ref,
                  g_ref, bta_ref, hnew_ref):
    z = (uh_ref[...] + a0_ref[...] + a1_ref[...]
         + b0_ref[...] + b1_ref[...])
    mu = jnp.mean(z, axis=0, keepdims=True)
    var = jnp.mean((z - mu) * (z - mu), axis=0, keepdims=True)
    zn = (z - mu) * lax.rsqrt(var + 1e-5) * g_ref[...] + bta_ref[...]
    hnew_ref[...] = h_ref[...] + zn


_node_pass = pl.pallas_call(
    _node_bn_body,
    out_shape=jax.ShapeDtypeStruct((_N, _D), jnp.float32),
)


@functools.lru_cache(maxsize=2)
def _make_gather(h):
    @functools.partial(
        pl.kernel,
        mesh=_sc_mesh(),
        out_type=jax.ShapeDtypeStruct((_EH, _D), jnp.float32),
        scratch_types=[
            pltpu.VMEM((_GCMAX, 1, _CH), jnp.int32),
            pltpu.VMEM((_GCMAX, 1, _CH), jnp.int32),
            pltpu.VMEM((_CH, _D), jnp.float32),
            pltpu.VMEM((_CH, _D), jnp.float32),
            pltpu.VMEM((_CH, _D), jnp.float32),
            pltpu.VMEM((_CH, _D), jnp.float32),
            pltpu.SemaphoreType.DMA,
            pltpu.SemaphoreType.DMA,
            pltpu.SemaphoreType.DMA,
            pltpu.SemaphoreType.DMA,
            pltpu.SemaphoreType.DMA,
            pltpu.SemaphoreType.DMA,
        ],
    )
    def _gather(qh, rh, src3, dst3, g1,
                src_i, dst_i, qb0, qb1, rb0, rb1,
                gq0, gq1, gr0, gr1, os0, os1):
        wid = lax.axis_index("s") * _NC + lax.axis_index("c")
        cnt = jnp.where(wid < 16, 63, 62)
        basec = wid * 62 + jnp.minimum(wid, 16)
        base0 = basec * _CH
        qbs, rbs = (qb0, qb1), (rb0, rb1)
        gqs, grs, oss = (gq0, gq1), (gr0, gr1), (os0, os1)
        pltpu.sync_copy(src3.at[pl.ds(h * 2000 + basec, _GCMAX)], src_i)
        pltpu.sync_copy(dst3.at[pl.ds(h * 2000 + basec, _GCMAX)], dst_i)
        pltpu.async_copy(qh.at[src_i.at[0, 0]], qbs[0], gqs[0])
        pltpu.async_copy(rh.at[dst_i.at[0, 0]], rbs[0], grs[0])

        @pl.loop(0, _GCMAX + 1, step=2)
        def pair(ii):
            for b in range(2):
                i = ii + b

                @pl.when(i < cnt)
                def _():
                    nb = 1 - b
                    @pl.when(i > 0)
                    def _():
                        pltpu.make_async_copy(
                            qbs[nb], g1.at[pl.ds(base0, _CH)], oss[nb]).wait()

                    @pl.when(i + 1 < cnt)
                    def _():
                        pltpu.async_copy(qh.at[src_i.at[i + 1, 0]], qbs[nb],
                                         gqs[nb])
                        pltpu.async_copy(rh.at[dst_i.at[i + 1, 0]], rbs[nb],
                                         grs[nb])

                    pltpu.make_async_copy(qh.at[src_i.at[i, 0]], qbs[b],
                                          gqs[b]).wait()
                    pltpu.make_async_copy(rh.at[dst_i.at[i, 0]], rbs[b],
                                          grs[b]).wait()

                    @pl.loop(0, _CH, unroll=4)
                    def row(r):
                        for g in range(_D // 16):
                            sl = pl.ds(g * 16, 16)
                            plsc.addupdate(qbs[b].at[r, sl], rbs[b][r, sl])

                    pltpu.async_copy(
                        qbs[b], g1.at[pl.ds(base0 + i * _CH, _CH)], oss[b])

        @pl.when(cnt == 63)
        def _():
            pltpu.make_async_copy(qbs[0], g1.at[pl.ds(base0, _CH)],
                                  oss[0]).wait()

        @pl.when(cnt == 62)
        def _():
            pltpu.make_async_copy(qbs[1], g1.at[pl.ds(base0, _CH)],
                                  oss[1]).wait()

    return _gather


@functools.lru_cache(maxsize=1)
def _make_scatter():
    @functools.partial(
        pl.kernel,
        mesh=_sc_mesh(),
        out_type=jax.ShapeDtypeStruct((_NC, _NP, _D), jnp.float32),
        scratch_types=[
            pltpu.VMEM((_CS,), jnp.int32),
            pltpu.VMEM((_CS,), jnp.int32),
            pltpu.VMEM((_CS,), jnp.int32),
            pltpu.VMEM((_CS,), jnp.int32),
            pltpu.VMEM((_CS, _D), jnp.float32),
            pltpu.VMEM((_CS, _D), jnp.float32),
            pltpu.VMEM((_CS, _D), jnp.float32),
            pltpu.VMEM_SHARED((_NP, _D), jnp.float32),
            pltpu.SemaphoreType.DMA,
            pltpu.SemaphoreType.DMA,
            pltpu.SemaphoreType.DMA,
            pltpu.SemaphoreType.DMA,
            pltpu.SemaphoreType.DMA,
            pltpu.SemaphoreType.DMA,
            pltpu.SemaphoreType.DMA,
        ],
    )
    def _scatter(gate, vh, srcf, dstf, zeros_hbm, out,
                 sv0, sv1, dv0, dv1, vb0, vb1, gb, agg_sh,
                 is0, is1, gv0, gv1, ss0, ss1, gbsem):
        c = lax.axis_index("c")
        s = lax.axis_index("s")
        wid = s * _NC + c
        svs, dvs, vbs = (sv0, sv1), (dv0, dv1), (vb0, vb1)
        iss, gvs, sss = (is0, is1), (gv0, gv1), (ss0, ss1)
        pltpu.sync_copy(zeros_hbm.at[pl.ds(s * _RPT, _RPT)],
                        agg_sh.at[pl.ds(s * _RPT, _RPT)])
        cnt = jnp.where(wid < 4, 79, 78)
        base0 = (wid * 78 + jnp.minimum(wid, 4)) * _CS
        pltpu.sync_copy(srcf.at[pl.ds(base0, _CS)], svs[0])
        pltpu.sync_copy(dstf.at[pl.ds(base0, _CS)], dvs[0])
        pltpu.async_copy(vh.at[dvs[0]], vbs[0], gvs[0])
        pltpu.sync_copy(gate.at[pl.ds(base0, _CS)], gb)
        plsc.subcore_barrier()

        @pl.loop(0, _SCMAX + 1, step=2)
        def pair(ii):
            for b in range(2):
                i = ii + b

                @pl.when(i < cnt)
                def _():
                    nb = 1 - b
                    base = base0 + i * _CS

                    @pl.when(i > 0)
                    def _():
                        pltpu.make_async_copy(
                            gate.at[pl.ds(base0, _CS)], vbs[nb],
                            sss[nb]).wait()

                    @pl.when(i + 1 < cnt)
                    def _():
                        pltpu.async_copy(
                            srcf.at[pl.ds(base + _CS, _CS)], svs[nb], iss[nb])
                        pltpu.async_copy(
                            dstf.at[pl.ds(base + _CS, _CS)], dvs[nb], iss[nb])

                    pltpu.make_async_copy(
                        gate.at[pl.ds(base0, _CS)], vbs[b], gvs[b]).wait()

                    @pl.when(i > 0)
                    def _():
                        pltpu.make_async_copy(
                            gate.at[pl.ds(base0, _CS)], gb, gbsem).wait()

                    @pl.when(i + 1 < cnt)
                    def _():
                        pltpu.make_async_copy(
                            srcf.at[pl.ds(base0, _CS)], svs[nb],
                            iss[nb]).wait()
                        pltpu.make_async_copy(
                            dstf.at[pl.ds(base0, _CS)], dvs[nb],
                            iss[nb]).wait()
                        pltpu.async_copy(vh.at[dvs[nb]], vbs[nb], gvs[nb])

                    @pl.loop(0, _CS, unroll=4)
                    def row(r):
                        for g in range(_D // 16):
                            sl = pl.ds(g * 16, 16)
                            vbs[b][r, sl] = gb[r, sl] * vbs[b][r, sl]

                    @pl.when(i + 1 < cnt)
                    def _():
                        pltpu.async_copy(gate.at[pl.ds(base + _CS, _CS)],
                                         gb, gbsem)

                    pltpu.async_copy(vbs[b], agg_sh.at[svs[b]], sss[b],
                                     add=True)

        @pl.when(cnt == 79)
        def _():
            pltpu.make_async_copy(
                gate.at[pl.ds(base0, _CS)], vbs[0], sss[0]).wait()

        @pl.when(cnt == 78)
        def _():
            pltpu.make_async_copy(
                gate.at[pl.ds(base0, _CS)], vbs[1], sss[1]).wait()

        plsc.subcore_barrier()
        pltpu.sync_copy(agg_sh.at[pl.ds(s * _RPT, _RPT)],
                        out.at[c, pl.ds(s * _RPT, _RPT)])

    return _scatter


def kernel(h, e, edge_index, P_w, Q_w, R_w, U_w, V_w,
           m1_w, m1_b, m2_w, m2_b, e_gamma, e_beta, n_gamma, n_beta):
    src = edge_index[0]
    dst = edge_index[1]
    src3 = jnp.pad(src, (0, 8 * _CH)).reshape(_E // _CH + 8, 1, _CH)
    dst3 = jnp.pad(dst, (0, 8 * _CH)).reshape(_E // _CH + 8, 1, _CH)
    w4 = jnp.concatenate([Q_w.T, R_w.T, V_w.T, U_w.T], axis=1)
    qh, rh, vh, uh = _node_mm(h, w4)
    zagg = jnp.zeros((_NP, _D), jnp.float32)

    g1a = _make_gather(0)(qh, rh, src3, dst3)
    g1b = _make_gather(1)(qh, rh, src3, dst3)
    ehat_a, gate_a, stats_a = _make_edge_pass1(0)(P_w.T, e, g1a)
    ehat_b, gate_b, stats_b = _make_edge_pass1(1)(P_w.T, e, g1b)
    scat = _make_scatter()
    agg_a = scat(gate_a, vh, src[:_EH], dst[:_EH], zagg)
    agg_b = scat(gate_b, vh, src[_EH:], dst[_EH:], zagg)

    mlp_args = (m1_w.T, m2_w.T, m1_b.reshape(1, _D), m2_b.reshape(1, _D),
                e_gamma.reshape(1, _D), e_beta.reshape(1, _D))
    en0 = _make_edge_pass2(0)(stats_a, stats_b, *mlp_args, e, ehat_a)
    e_new = _make_edge_pass2(1)(en0, stats_a, stats_b, *mlp_args, e, ehat_b)
    h_new = _node_pass(h, uh, agg_a[0, :_N], agg_a[1, :_N],
                       agg_b[0, :_N], agg_b[1, :_N],
                       n_gamma.reshape(1, _D), n_beta.reshape(1, _D))
    return (h_new, e_new)
